# baseline (speedup 1.0000x reference)
import jax
import jax.numpy as jnp
from jax import lax
from jax.experimental import pallas as pl
from jax.experimental.pallas import tpu as pltpu
from jax.experimental.pallas import tpu_sc as plsc

N = 10000
D = 128
DE = 16
E = 320000
ITERS = 2

NC = 2
NS = 16
NW = NC * NS
NP = 10112
CH = 32
CPT = 320
SLOTS = 4
IB = 32
BPT = CPT // IB
EPT = CH * CPT
EP = NW * EPT
RPT = NP // NS
BLK = 1264


def _sc_scatter_body(gather_rows, *refs):
    (feat_hbm, idx_hbm, out_hbm,
     idxp, rows, acc_sh,
     ga0, ga1, ga2, ga3, gb0, gb1, gb2, gb3, ss) = refs
    ga_sems = (ga0, ga1, ga2, ga3)
    gb_sems = (gb0, gb1, gb2, gb3)

    cid = lax.axis_index("c")
    sid = lax.axis_index("s")
    tid = cid * NS + sid
    r0 = sid * RPT

    zero16 = jnp.zeros((16,), jnp.float32)

    def _fill_z(i, _):
        for j in range(D // 16):
            rows[0, 0, i, pl.ds(j * 16, 16)] = zero16
        return 0

    lax.fori_loop(0, CH, _fill_z, 0)

    def _init(cblk, _):
        pltpu.sync_copy(rows.at[0, 0],
                        acc_sh.at[pl.ds(r0 + cblk * CH, CH)])
        return 0

    lax.fori_loop(0, RPT // CH, _init, 0)
    pltpu.sync_copy(rows.at[0, 0, pl.ds(0, RPT % CH)],
                    acc_sh.at[pl.ds(r0 + (RPT // CH) * CH, RPT % CH)])

    plsc.subcore_barrier()

    ebase = tid * EPT

    def _gather_start(c, slot):
        pos = lax.rem(c, 2 * IB)
        if gather_rows:
            pltpu.async_copy(feat_hbm.at[idxp.at[pos, 1]], rows.at[0, slot],
                             ga_sems[slot])
            pltpu.async_copy(feat_hbm.at[idxp.at[pos, 0]], rows.at[1, slot],
                             gb_sems[slot])
        else:
            pltpu.async_copy(feat_hbm.at[pl.ds(ebase + c * CH, CH)],
                             rows.at[0, slot], ga_sems[slot])

    def _gather_wait(slot):
        pltpu.make_async_copy(feat_hbm.at[pl.ds(0, CH)], rows.at[0, slot],
                              ga_sems[slot]).wait()
        if gather_rows:
            pltpu.make_async_copy(feat_hbm.at[pl.ds(0, CH)], rows.at[1, slot],
                                  gb_sems[slot]).wait()

    pltpu.sync_copy(idx_hbm.at[tid, pl.ds(0, IB)], idxp.at[pl.ds(0, IB)])
    for s in range(SLOTS):
        _gather_start(s, s)

    def _block(b, _):
        bnext = jnp.minimum(b + 1, BPT - 1)
        half = lax.rem(bnext, 2) * IB
        pltpu.sync_copy(idx_hbm.at[tid, pl.ds(bnext * IB, IB)],
                        idxp.at[pl.ds(half, IB)])

        def _group(j, _):
            for slot in range(SLOTS):
                c = b * IB + SLOTS * j + slot
                pos = lax.rem(c, 2 * IB)
                _gather_wait(slot)
                src_idx = idxp.at[pos, 0]
                dst_idx = idxp.at[pos, 1]
                if gather_rows:
                    sc1 = pltpu.async_copy(rows.at[0, slot],
                                           acc_sh.at[src_idx], ss, add=True)
                    sc2 = pltpu.async_copy(rows.at[1, slot],
                                           acc_sh.at[dst_idx], ss, add=True)
                else:
                    sc1 = pltpu.async_copy(rows.at[0, slot],
                                           acc_sh.at[src_idx], ss, add=True)
                    sc2 = pltpu.async_copy(rows.at[0, slot],
                                           acc_sh.at[dst_idx], ss, add=True)
                sc1.wait()
                sc2.wait()
                _gather_start(jnp.minimum(c + SLOTS, CPT - 1), slot)
            return 0

        lax.fori_loop(0, IB // SLOTS, _group, 0)
        return 0

    lax.fori_loop(0, BPT, _block, 0)

    for s in range(SLOTS):
        _gather_wait(s)

    plsc.subcore_barrier()

    pltpu.sync_copy(acc_sh.at[pl.ds(r0, RPT)], out_hbm.at[cid, pl.ds(r0, RPT)])


_SC_MESH = plsc.VectorSubcoreMesh(
    core_axis_name="c", subcore_axis_name="s", num_cores=NC, num_subcores=NS)

_SC_OUT = [jax.ShapeDtypeStruct((NC, NP, D), jnp.float32)]

_SC_SCRATCH = [
    pltpu.VMEM((2 * IB, 2, CH), jnp.int32),
    pltpu.VMEM((2, SLOTS, CH, D), jnp.float32),
    pltpu.VMEM_SHARED((NP, D), jnp.float32),
] + [pltpu.SemaphoreType.DMA] * (2 * SLOTS + 1)

_sc_agg = pl.kernel(
    lambda *refs: _sc_scatter_body(True, *refs),
    out_type=_SC_OUT, mesh=_SC_MESH, scratch_types=_SC_SCRATCH)

_sc_efeat = pl.kernel(
    lambda *refs: _sc_scatter_body(False, *refs),
    out_type=_SC_OUT, mesh=_SC_MESH, scratch_types=_SC_SCRATCH)


def _emb_body(x_ref, w_ref, b_ref, o_ref):
    z = jnp.dot(x_ref[...], w_ref[...],
                preferred_element_type=jnp.float32) + b_ref[...]
    o_ref[...] = jnp.maximum(z, 0.0)


_emb_call = pl.pallas_call(
    _emb_body,
    grid=(NP // BLK,),
    in_specs=[
        pl.BlockSpec((BLK, D), lambda i: (i, 0)),
        pl.BlockSpec((D, D), lambda i: (0, 0)),
        pl.BlockSpec((1, D), lambda i: (0, 0)),
    ],
    out_specs=pl.BlockSpec((BLK, D), lambda i: (i, 0)),
    out_shape=jax.ShapeDtypeStruct((NP, D), jnp.float32),
)


def _upd_body(h_ref, a0_ref, a1_ref, m0_ref, m1_ref, w_ref, w16_ref, b_ref,
              o_ref):
    x = h_ref[...] + a0_ref[0] + a1_ref[0]
    em = m0_ref[0] + m1_ref[0]
    dg = em[:, DE:DE + 1] + 1.0
    x = x / dg
    e = em[:, 0:DE]
    z = (jnp.dot(x, w_ref[...], preferred_element_type=jnp.float32)
         + jnp.dot(e, w16_ref[...], preferred_element_type=jnp.float32)
         + b_ref[...])
    o_ref[...] = 1.0 / (1.0 + jnp.exp(-z))


_upd_call = pl.pallas_call(
    _upd_body,
    grid=(NP // BLK,),
    in_specs=[
        pl.BlockSpec((BLK, D), lambda i: (i, 0)),
        pl.BlockSpec((1, BLK, D), lambda i: (0, i, 0)),
        pl.BlockSpec((1, BLK, D), lambda i: (1, i, 0)),
        pl.BlockSpec((1, BLK, D), lambda i: (0, i, 0)),
        pl.BlockSpec((1, BLK, D), lambda i: (1, i, 0)),
        pl.BlockSpec((D, D), lambda i: (0, 0)),
        pl.BlockSpec((DE, D), lambda i: (0, 0)),
        pl.BlockSpec((1, D), lambda i: (0, 0)),
    ],
    out_specs=pl.BlockSpec((BLK, D), lambda i: (i, 0)),
    out_shape=jax.ShapeDtypeStruct((NP, D), jnp.float32),
)


def kernel(inputs, edges, adjacency, membership, W_emb, b_emb, W_upd, b_upd):
    del membership
    src = adjacency[:, 0]
    dst = adjacency[:, 1]
    pad_e = EP - E
    src_p = jnp.concatenate(
        [src, jnp.full((pad_e,), N, jnp.int32)]).reshape(NW, CPT, CH)
    dst_p = jnp.concatenate(
        [dst, jnp.full((pad_e,), N, jnp.int32)]).reshape(NW, CPT, CH)
    idx_p = jnp.stack([src_p, dst_p], axis=2)
    efeat_p = jnp.concatenate(
        [edges, jnp.ones((E, 1), jnp.float32),
         jnp.zeros((E, D - DE - 1), jnp.float32)], axis=1)
    efeat_p = jnp.concatenate(
        [efeat_p, jnp.zeros((pad_e, D), jnp.float32)], axis=0)
    x_p = jnp.concatenate([inputs, jnp.zeros((NP - N, D), jnp.float32)])
    b_emb2 = b_emb.reshape(1, D)
    b_upd2 = b_upd.reshape(1, D)
    w16 = W_upd[:DE]

    h = _emb_call(x_p, W_emb, b_emb2)
    (em,) = _sc_efeat(efeat_p, idx_p)
    for _ in range(ITERS):
        (agg,) = _sc_agg(h, idx_p)
        h = _upd_call(h, agg, agg, em, em, W_upd, w16, b_upd2)
    return h[:N]


# speedup vs baseline: 1.1060x; 1.1060x over previous
import jax
import jax.numpy as jnp
from jax import lax
from jax.experimental import pallas as pl
from jax.experimental.pallas import tpu as pltpu
from jax.experimental.pallas import tpu_sc as plsc

N = 10000
D = 128
DE = 16
E = 320000
ITERS = 2

NC = 2
NS = 16
NW = NC * NS
NP = 10112
CH = 64
CPT = 160
IB = 16
BPT = CPT // IB
EPT = CH * CPT
EP = NW * EPT
RPT = NP // NS
BLK = 1264


def _sc_scatter_body(gather_rows, cpt_by_core, *refs):
    (feat_hbm, idx_hbm, out_hbm,
     idxp, rows, acc_sh,
     ga0, ga1, gb0, gb1, ss) = refs
    ga_sems = (ga0, ga1)
    gb_sems = (gb0, gb1)

    cpt0, cpt1 = cpt_by_core
    cid = lax.axis_index("c")
    sid = lax.axis_index("s")
    r0 = sid * RPT
    start = jnp.where(cid == 0, sid * cpt0, NS * cpt0 + sid * cpt1)
    nchunks = jnp.where(cid == 0, cpt0, cpt1)
    nblocks = nchunks // IB

    zero16 = jnp.zeros((16,), jnp.float32)

    def _fill_z(i, _):
        for j in range(D // 16):
            rows[0, 0, i, pl.ds(j * 16, 16)] = zero16
        return 0

    lax.fori_loop(0, CH, _fill_z, 0)

    def _init(cblk, _):
        pltpu.sync_copy(rows.at[0, 0],
                        acc_sh.at[pl.ds(r0 + cblk * CH, CH)])
        return 0

    lax.fori_loop(0, RPT // CH, _init, 0)
    pltpu.sync_copy(rows.at[0, 0, pl.ds(0, RPT % CH)],
                    acc_sh.at[pl.ds(r0 + (RPT // CH) * CH, RPT % CH)])

    plsc.subcore_barrier()

    def _gather_start(c, slot):
        pos = lax.rem(c, 2 * IB)
        if gather_rows:
            pltpu.async_copy(feat_hbm.at[idxp.at[pos, 1]], rows.at[0, slot],
                             ga_sems[slot])
            pltpu.async_copy(feat_hbm.at[idxp.at[pos, 0]], rows.at[1, slot],
                             gb_sems[slot])
        else:
            pltpu.async_copy(feat_hbm.at[pl.ds((start + c) * CH, CH)],
                             rows.at[0, slot], ga_sems[slot])

    def _gather_wait(slot):
        pltpu.make_async_copy(feat_hbm.at[pl.ds(0, CH)], rows.at[0, slot],
                              ga_sems[slot]).wait()
        if gather_rows:
            pltpu.make_async_copy(feat_hbm.at[pl.ds(0, CH)], rows.at[1, slot],
                                  gb_sems[slot]).wait()

    pltpu.sync_copy(idx_hbm.at[pl.ds(start, IB)], idxp.at[pl.ds(0, IB)])
    _gather_start(0, 0)
    _gather_start(1, 1)

    def _block(b, _):
        bnext = jnp.minimum(b + 1, nblocks - 1)
        half = lax.rem(bnext, 2) * IB
        pltpu.sync_copy(idx_hbm.at[pl.ds(start + bnext * IB, IB)],
                        idxp.at[pl.ds(half, IB)])

        def _pair(j, _):
            for slot in range(2):
                c = b * IB + 2 * j + slot
                pos = lax.rem(c, 2 * IB)
                _gather_wait(slot)
                src_idx = idxp.at[pos, 0]
                dst_idx = idxp.at[pos, 1]
                if gather_rows:
                    sc1 = pltpu.async_copy(rows.at[0, slot],
                                           acc_sh.at[src_idx], ss, add=True)
                    sc2 = pltpu.async_copy(rows.at[1, slot],
                                           acc_sh.at[dst_idx], ss, add=True)
                else:
                    sc1 = pltpu.async_copy(rows.at[0, slot],
                                           acc_sh.at[src_idx], ss, add=True)
                    sc2 = pltpu.async_copy(rows.at[0, slot],
                                           acc_sh.at[dst_idx], ss, add=True)
                sc1.wait()
                sc2.wait()
                _gather_start(jnp.minimum(c + 2, nchunks - 1), slot)
            return 0

        lax.fori_loop(0, IB // 2, _pair, 0)
        return 0

    lax.fori_loop(0, nblocks, _block, 0)

    _gather_wait(0)
    _gather_wait(1)

    plsc.subcore_barrier()

    pltpu.sync_copy(acc_sh.at[pl.ds(r0, RPT)], out_hbm.at[cid, pl.ds(r0, RPT)])


_SC_MESH = plsc.VectorSubcoreMesh(
    core_axis_name="c", subcore_axis_name="s", num_cores=NC, num_subcores=NS)

_SC_OUT = [jax.ShapeDtypeStruct((NC, NP, D), jnp.float32)]

_SC_SCRATCH = [
    pltpu.VMEM((2 * IB, 2, CH), jnp.int32),
    pltpu.VMEM((2, 2, CH, D), jnp.float32),
    pltpu.VMEM_SHARED((NP, D), jnp.float32),
    pltpu.SemaphoreType.DMA,
    pltpu.SemaphoreType.DMA,
    pltpu.SemaphoreType.DMA,
    pltpu.SemaphoreType.DMA,
    pltpu.SemaphoreType.DMA,
]

AGG_SPLIT = (240, 80)
EF_SPLIT = (160, 160)

_sc_agg = pl.kernel(
    lambda *refs: _sc_scatter_body(True, AGG_SPLIT, *refs),
    out_type=_SC_OUT, mesh=_SC_MESH, scratch_types=_SC_SCRATCH)

_sc_efeat = pl.kernel(
    lambda *refs: _sc_scatter_body(False, EF_SPLIT, *refs),
    out_type=_SC_OUT, mesh=_SC_MESH, scratch_types=_SC_SCRATCH)


def _emb_body(x_ref, w_ref, b_ref, o_ref):
    z = jnp.dot(x_ref[...], w_ref[...],
                preferred_element_type=jnp.float32) + b_ref[...]
    o_ref[...] = jnp.maximum(z, 0.0)


_emb_call = pl.pallas_call(
    _emb_body,
    grid=(NP // BLK,),
    in_specs=[
        pl.BlockSpec((BLK, D), lambda i: (i, 0)),
        pl.BlockSpec((D, D), lambda i: (0, 0)),
        pl.BlockSpec((1, D), lambda i: (0, 0)),
    ],
    out_specs=pl.BlockSpec((BLK, D), lambda i: (i, 0)),
    out_shape=jax.ShapeDtypeStruct((NP, D), jnp.float32),
)


def _upd_body(h_ref, a0_ref, a1_ref, m0_ref, m1_ref, w_ref, w16_ref, b_ref,
              o_ref):
    x = h_ref[...] + a0_ref[0] + a1_ref[0]
    em = m0_ref[0] + m1_ref[0]
    dg = em[:, DE:DE + 1] + 1.0
    x = x / dg
    e = em[:, 0:DE]
    z = (jnp.dot(x, w_ref[...], preferred_element_type=jnp.float32)
         + jnp.dot(e, w16_ref[...], preferred_element_type=jnp.float32)
         + b_ref[...])
    o_ref[...] = 1.0 / (1.0 + jnp.exp(-z))


_upd_call = pl.pallas_call(
    _upd_body,
    grid=(NP // BLK,),
    in_specs=[
        pl.BlockSpec((BLK, D), lambda i: (i, 0)),
        pl.BlockSpec((1, BLK, D), lambda i: (0, i, 0)),
        pl.BlockSpec((1, BLK, D), lambda i: (1, i, 0)),
        pl.BlockSpec((1, BLK, D), lambda i: (0, i, 0)),
        pl.BlockSpec((1, BLK, D), lambda i: (1, i, 0)),
        pl.BlockSpec((D, D), lambda i: (0, 0)),
        pl.BlockSpec((DE, D), lambda i: (0, 0)),
        pl.BlockSpec((1, D), lambda i: (0, 0)),
    ],
    out_specs=pl.BlockSpec((BLK, D), lambda i: (i, 0)),
    out_shape=jax.ShapeDtypeStruct((NP, D), jnp.float32),
)


def kernel(inputs, edges, adjacency, membership, W_emb, b_emb, W_upd, b_upd):
    del membership
    src = adjacency[:, 0]
    dst = adjacency[:, 1]
    pad_e = EP - E
    src_p = jnp.concatenate(
        [src, jnp.full((pad_e,), N, jnp.int32)]).reshape(NW, CPT, CH)
    dst_p = jnp.concatenate(
        [dst, jnp.full((pad_e,), N, jnp.int32)]).reshape(NW, CPT, CH)
    idx_p = jnp.stack([src_p, dst_p], axis=2).reshape(-1, 2, CH)
    efeat_p = jnp.concatenate(
        [edges, jnp.ones((E, 1), jnp.float32),
         jnp.zeros((E, D - DE - 1), jnp.float32)], axis=1)
    efeat_p = jnp.concatenate(
        [efeat_p, jnp.zeros((pad_e, D), jnp.float32)], axis=0)
    x_p = jnp.concatenate([inputs, jnp.zeros((NP - N, D), jnp.float32)])
    b_emb2 = b_emb.reshape(1, D)
    b_upd2 = b_upd.reshape(1, D)
    w16 = W_upd[:DE]

    h = _emb_call(x_p, W_emb, b_emb2)
    (em,) = _sc_efeat(efeat_p, idx_p)
    for _ in range(ITERS):
        (agg,) = _sc_agg(h, idx_p)
        h = _upd_call(h, agg, agg, em, em, W_upd, w16, b_upd2)
    return h[:N]
